# Initial kernel scaffold; baseline (speedup 1.0000x reference)
import jax
import jax.numpy as jnp
from jax.experimental import pallas as pl


def kernel(x, edge_index, edge_attr, params):
    raise NotImplementedError("write your pallas kernel here")


# speedup vs baseline: 4.7856x; 4.7856x over previous
import functools

import jax
import jax.numpy as jnp
from jax import lax
from jax.experimental import pallas as pl
from jax.experimental.pallas import tpu as pltpu
from jax.experimental.pallas import tpu_sc as plsc

F32 = jnp.float32
I32 = jnp.int32

_NC = 2
_NS = 16
_NW = _NC * _NS
_CH = 1024


def _ceil_to(v, m):
    return (v + m - 1) // m * m


def _pick_block(n, cands=(6400, 3200, 2000, 1600, 1024, 800, 512, 256, 128, 64, 32, 16, 8)):
    for c in cands:
        if n % c == 0:
            return c
    return n


def _full_spec(arr):
    nd = arr.ndim
    return pl.BlockSpec(arr.shape, lambda i, _nd=nd: (0,) * _nd)


_PREC = lax.Precision.DEFAULT


def _mlp(h, ws, bs, ln=None):
    n = len(ws)
    for i in range(n):
        h = jnp.dot(h, ws[i], preferred_element_type=F32, precision=_PREC) + bs[i]
        if i < n - 1:
            h = jnp.maximum(h, 0.0)
    if ln is not None:
        h = _ln_post(h, ln[0], ln[1])
    return h


def _ln_post(xc, g, be):
    ones = jnp.ones((xc.shape[1], 1), dtype=F32)
    s = jnp.dot(xc * xc, ones, preferred_element_type=F32, precision=_PREC) * (1.0 / xc.shape[1])
    return xc / jnp.sqrt(s + 1e-5) * g + be


def _prep_params(p):
    ws = list(p["Ws"])
    bs = [b.reshape(1, -1) for b in p["bs"]]
    ln = None
    if "ln_g" in p:
        d = ws[-1].shape[1]
        c = jnp.eye(d, dtype=F32) - jnp.full((d, d), 1.0 / d, dtype=F32)
        ws[-1] = ws[-1] @ c
        bs[-1] = bs[-1] @ c
        ln = (p["ln_g"].reshape(1, -1), p["ln_b"].reshape(1, -1))
    return ws, bs, ln


def _flatten_wb(ws, bs, ln):
    out = list(ws) + list(bs)
    if ln is not None:
        out += [ln[0], ln[1]]
    return out


def _unflatten_wb(args, n_layers, has_ln):
    ws = list(args[:n_layers])
    bs = list(args[n_layers:2 * n_layers])
    ln = None
    rest = args[2 * n_layers:]
    if has_ln:
        ln = (rest[0], rest[1])
        rest = rest[2:]
    return ws, bs, ln, rest


def _tc_node_stats(x):
    n, d = x.shape
    nrow = n * d // 128
    nrow_pad = _ceil_to(nrow, 8)
    xf = jnp.concatenate(
        [x.reshape(-1), jnp.zeros((nrow_pad * 128 - n * d,), dtype=F32)]
    ).reshape(nrow_pad, 128)
    b = _pick_block(nrow_pad, (1024, 512, 256, 136, 128, 64, 32, 16, 8))

    def body(x_ref, o_ref):
        i = pl.program_id(0)
        xb = x_ref[...]
        s1 = jnp.sum(xb, axis=0, keepdims=True)
        s2 = jnp.sum(xb * xb, axis=0, keepdims=True)
        blk = jnp.concatenate(
            [s1, s2, jnp.zeros((6, 128), dtype=F32)], axis=0)

        @pl.when(i == 0)
        def _():
            o_ref[...] = blk

        @pl.when(i > 0)
        def _():
            o_ref[...] = o_ref[...] + blk

    sums = pl.pallas_call(
        body,
        grid=(nrow_pad // b,),
        in_specs=[pl.BlockSpec((b, 128), lambda i: (i, 0))],
        out_specs=pl.BlockSpec((8, 128), lambda i: (0, 0)),
        out_shape=jax.ShapeDtypeStruct((8, 128), F32),
    )(xf)
    s1 = sums[0].reshape(128 // d, d).sum(axis=0)
    s2 = sums[1].reshape(128 // d, d).sum(axis=0)
    mean = s1 / n
    var = s2 / n - mean * mean
    std = jnp.maximum(jnp.sqrt(jnp.maximum(var, 0.0)), 1e-8)
    return mean.reshape(1, d), std.reshape(1, d)


def _tc_edge_stats(edge_attr):
    e = edge_attr.shape[0]
    nrow = e * 2 // 128
    eaf = edge_attr.reshape(nrow, 128)
    b = _pick_block(nrow, (1000, 512, 256, 128, 64, 32, 16, 8))
    pr = jnp.zeros((128, 128), dtype=F32)
    ii = jnp.arange(64)
    pr = pr.at[2 * ii, 2 * ii].set(1.0).at[2 * ii + 1, 2 * ii].set(1.0)

    def body(x_ref, p_ref, o_ref):
        i = pl.program_id(0)
        xb = x_ref[...]
        sq = xb * xb
        d2 = jnp.dot(sq, p_ref[...], preferred_element_type=F32, precision=_PREC)
        dist = jnp.sqrt(d2)
        blk = jnp.concatenate(
            [jnp.sum(xb, axis=0, keepdims=True),
             jnp.sum(sq, axis=0, keepdims=True),
             jnp.sum(dist, axis=0, keepdims=True),
             jnp.sum(d2, axis=0, keepdims=True),
             jnp.zeros((4, 128), dtype=F32)], axis=0)

        @pl.when(i == 0)
        def _():
            o_ref[...] = blk

        @pl.when(i > 0)
        def _():
            o_ref[...] = o_ref[...] + blk

    sums = pl.pallas_call(
        body,
        grid=(nrow // b,),
        in_specs=[pl.BlockSpec((b, 128), lambda i: (i, 0)), _full_spec(pr)],
        out_specs=pl.BlockSpec((8, 128), lambda i: (0, 0)),
        out_shape=jax.ShapeDtypeStruct((8, 128), F32),
    )(eaf, pr)
    s_ea = sums[0].reshape(64, 2).sum(axis=0)
    s_sq = sums[1].reshape(64, 2).sum(axis=0)
    s_d = sums[2].reshape(64, 2).sum(axis=0)[0]
    s_d2 = sums[3].reshape(64, 2).sum(axis=0)[0]
    s1 = jnp.concatenate([s_ea, s_d[None]])
    s2 = jnp.concatenate([s_sq, s_d2[None]])
    mean = s1 / e
    var = s2 / e - mean * mean
    std = jnp.maximum(jnp.sqrt(jnp.maximum(var, 0.0)), 1e-8)
    return mean.reshape(1, 3), std.reshape(1, 3)


def _tc_encode_nodes(x_pad, mean_x, std_x, enc_w):
    n_pad, d = x_pad.shape
    b = 1024
    ws, bs, ln = enc_w
    n_layers = len(ws)
    wb = _flatten_wb(ws, bs, ln)

    def body(x_ref, mu_ref, sd_ref, *rest):
        o_ref = rest[-1]
        ws_, bs_, ln_, _ = _unflatten_wb([r[...] for r in rest[:-1]], n_layers, True)
        xb = x_ref[...]
        xn = (xb - mu_ref[...]) / sd_ref[...]
        h = _mlp(xn, ws_, bs_, ln_)
        zeros = jnp.zeros((b, 14), dtype=F32)
        o_ref[...] = jnp.concatenate([h, xb[:, 0:2], zeros], axis=1)

    return pl.pallas_call(
        body,
        grid=(n_pad // b,),
        in_specs=[pl.BlockSpec((b, d), lambda i: (i, 0)),
                  _full_spec(mean_x), _full_spec(std_x)]
                 + [_full_spec(w) for w in wb],
        out_specs=pl.BlockSpec((b, 32), lambda i: (i, 0)),
        out_shape=jax.ShapeDtypeStruct((n_pad, 32), F32),
    )(x_pad, mean_x, std_x, *wb)


def _sc_gather(table, idx2_row, idx2_col):
    n_pad, w_dim = table.shape
    e_pad = idx2_row.shape[0] * 128
    ch = 512
    epw = e_pad // _NW
    nch = epw // ch
    npair = nch // 2
    sub = ch // 128

    mesh = plsc.VectorSubcoreMesh(core_axis_name="c", subcore_axis_name="s",
                                  num_cores=_NC, num_subcores=_NS)

    @functools.partial(
        pl.kernel,
        out_type=(jax.ShapeDtypeStruct((e_pad, w_dim), F32),
                  jax.ShapeDtypeStruct((e_pad, w_dim), F32)),
        mesh=mesh,
        scratch_types=[pltpu.VMEM((sub, 128), I32), pltpu.VMEM((ch, w_dim), F32),
                       pltpu.VMEM((sub, 128), I32), pltpu.VMEM((ch, w_dim), F32),
                       pltpu.VMEM((sub, 128), I32), pltpu.VMEM((ch, w_dim), F32),
                       pltpu.VMEM((sub, 128), I32), pltpu.VMEM((ch, w_dim), F32),
                       pltpu.SemaphoreType.DMA, pltpu.SemaphoreType.DMA,
                       pltpu.SemaphoreType.DMA, pltpu.SemaphoreType.DMA],
        compiler_params=pltpu.CompilerParams(use_tc_tiling_on_sc=False),
    )
    def k(table_h, row_h, col_h, osrc_h, odst_h,
          idx_ra, buf_ra, idx_ca, buf_ca, idx_rb, buf_rb, idx_cb, buf_cb,
          gsem_a, gsem_b, wsem_a, wsem_b):
        wid = lax.axis_index("s") * _NC + lax.axis_index("c")
        base = wid * epw

        def chunk_off(i):
            b0 = pl.multiple_of(base + i * ch, ch)
            return b0, pl.multiple_of(b0 // 128, sub)

        def gathers(idx_r, idx_c, buf_r, buf_c, gsem, b0, q0):
            pltpu.sync_copy(row_h.at[pl.ds(q0, sub)], idx_r)
            pltpu.sync_copy(col_h.at[pl.ds(q0, sub)], idx_c)
            cps = []
            for j in range(sub):
                cps.append(pltpu.async_copy(
                    table_h.at[idx_r.at[j]], buf_r.at[pl.ds(j * 128, 128)], gsem))
            for j in range(sub):
                cps.append(pltpu.async_copy(
                    table_h.at[idx_c.at[j]], buf_c.at[pl.ds(j * 128, 128)], gsem))
            return cps

        def wait_writeback(buf_r, buf_c, wsem):
            pltpu.make_async_copy(buf_r, osrc_h.at[pl.ds(0, ch)], wsem).wait()
            pltpu.make_async_copy(buf_c, odst_h.at[pl.ds(0, ch)], wsem).wait()

        def pair(j, carry):
            i0 = 2 * j
            b0a, q0a = chunk_off(i0)
            b0b, q0b = chunk_off(i0 + 1)

            @pl.when(j > 0)
            def _():
                wait_writeback(buf_ra, buf_ca, wsem_a)

            cps_a = gathers(idx_ra, idx_ca, buf_ra, buf_ca, gsem_a, b0a, q0a)

            @pl.when(j > 0)
            def _():
                wait_writeback(buf_rb, buf_cb, wsem_b)

            cps_b = gathers(idx_rb, idx_cb, buf_rb, buf_cb, gsem_b, b0b, q0b)
            for cp in cps_a:
                cp.wait()
            pltpu.async_copy(buf_ra, osrc_h.at[pl.ds(b0a, ch)], wsem_a)
            pltpu.async_copy(buf_ca, odst_h.at[pl.ds(b0a, ch)], wsem_a)
            for cp in cps_b:
                cp.wait()
            pltpu.async_copy(buf_rb, osrc_h.at[pl.ds(b0b, ch)], wsem_b)
            pltpu.async_copy(buf_cb, odst_h.at[pl.ds(b0b, ch)], wsem_b)
            return carry

        lax.fori_loop(0, npair, pair, 0)
        if nch % 2 == 1:
            i_last = nch - 1
            b0l, q0l = chunk_off(i_last)
            if npair > 0:
                wait_writeback(buf_ra, buf_ca, wsem_a)
            cps = gathers(idx_ra, idx_ca, buf_ra, buf_ca, gsem_a, b0l, q0l)
            for cp in cps:
                cp.wait()
            pltpu.async_copy(buf_ra, osrc_h.at[pl.ds(b0l, ch)], wsem_a)
            pltpu.async_copy(buf_ca, odst_h.at[pl.ds(b0l, ch)], wsem_a)
        if npair > 0 or nch % 2 == 1:
            wait_writeback(buf_ra, buf_ca, wsem_a)
        if npair > 0:
            wait_writeback(buf_rb, buf_cb, wsem_b)

    return k(table, idx2_row, idx2_col)


def _sc_scatter_add(payload, idx2_col, n_pad, zeros_rows, ch=_CH):
    e_pad, w_dim = payload.shape
    epw = e_pad // _NW
    nch = epw // ch
    sub = ch // 128
    rps = n_pad // _NS

    mesh = plsc.VectorSubcoreMesh(core_axis_name="c", subcore_axis_name="s",
                                  num_cores=_NC, num_subcores=_NS)

    @functools.partial(
        pl.kernel,
        out_type=jax.ShapeDtypeStruct((2 * n_pad, w_dim), F32),
        mesh=mesh,
        scratch_types=[pltpu.VMEM((sub, 128), I32),
                       pltpu.VMEM((ch, w_dim), F32),
                       pltpu.VMEM_SHARED((n_pad, w_dim), F32),
                       pltpu.SemaphoreType.DMA],
        compiler_params=pltpu.CompilerParams(use_tc_tiling_on_sc=False),
    )
    def k(pay_h, col_h, z_h, out_h, idx_c, buf, acc, sem):
        c = lax.axis_index("c")
        s = lax.axis_index("s")
        wid = s * _NC + c
        r0 = pl.multiple_of(s * rps, rps)
        pltpu.sync_copy(z_h, acc.at[pl.ds(r0, rps)])
        plsc.subcore_barrier()
        base = wid * epw

        def chunk(i, carry):
            b0 = pl.multiple_of(base + i * ch, ch)
            q0 = pl.multiple_of(b0 // 128, sub)
            pltpu.sync_copy(col_h.at[pl.ds(q0, sub)], idx_c)
            pltpu.sync_copy(pay_h.at[pl.ds(b0, ch)], buf)
            cps = []
            for j in range(sub):
                cps.append(pltpu.async_copy(
                    buf.at[pl.ds(j * 128, 128)],
                    acc.at[idx_c.at[j]], sem, add=True))
            for cp in cps:
                cp.wait()
            return carry

        lax.fori_loop(0, nch, chunk, 0)
        plsc.subcore_barrier()
        o0 = pl.multiple_of(c * n_pad + s * rps, rps)
        pltpu.sync_copy(acc.at[pl.ds(r0, rps)], out_h.at[pl.ds(o0, rps)])

    return k(payload, idx2_col, zeros_rows)


def _rep_blockdiag(w, r):
    k, n = w.shape
    out = jnp.zeros((r * k, r * n), dtype=F32)
    for q in range(r):
        out = out.at[q * k:(q + 1) * k, q * n:(q + 1) * n].set(w)
    return out


def _rep_row(b, r):
    return jnp.tile(b, (1, r))


def _tc_edge_pass1(gsrc, gdst, ea_pad, mean_e, std_e, light_w, ence_w, proce_w):
    e_pad = gsrc.shape[0]
    rows = e_pad // 4
    rb = 512
    lws, lbs, _ = light_w
    ews, ebs, eln = ence_w
    pws, pbs, pln = proce_w
    inv_s = (1.0 / std_e).reshape(3, 1)
    mu = mean_e.reshape(1, 3)

    a_ea = jnp.concatenate(
        [ews[0][0:2] * inv_s[0:2], lws[0][2:4]], axis=1)
    a_di = jnp.concatenate(
        [ews[0][2:3] * inv_s[2:3], lws[0][4:5]], axis=1)
    a_gs = jnp.zeros((32, 64), dtype=F32).at[16, 32:].set(lws[0][0])
    a_gd = (jnp.zeros((32, 64), dtype=F32)
            .at[16, 32:].set(-lws[0][0]).at[17, 32:].set(lws[0][1]))
    b_a = jnp.concatenate(
        [ebs[0] - (mu * inv_s.reshape(1, 3)) @ ews[0], lbs[0]], axis=1)
    w_b = _blockdiag([ews[1], lws[1]])
    b_b = jnp.concatenate([ebs[1], lbs[1]], axis=1)
    w_ce = jnp.concatenate(
        [ews[2], jnp.zeros((32, 16), dtype=F32)], axis=0)
    b_ce = ebs[2]
    w_cw = jnp.concatenate(
        [jnp.zeros((32, 2), dtype=F32), lws[2]], axis=0)
    b_cw = lbs[2]

    r4 = 4
    m_aea = _rep_blockdiag(a_ea, r4)
    m_adi = _rep_blockdiag(a_di, r4)
    m_ags = _rep_blockdiag(a_gs, r4)
    m_agd = _rep_blockdiag(a_gd, r4)
    v_ba = _rep_row(b_a, r4)
    m_b = _rep_blockdiag(w_b, r4)
    v_bb = _rep_row(b_b, r4)
    m_ce = _rep_blockdiag(w_ce, r4)
    v_bce = _rep_row(b_ce, r4)
    m_cw = _rep_blockdiag(w_cw, r4)
    v_bcw = _rep_row(b_cw, r4)
    m_j1 = _rep_blockdiag(jnp.full((16, 16), 1.0 / 16.0, dtype=F32), r4)
    v_g1 = _rep_row(eln[0], r4)
    v_be1 = _rep_row(eln[1], r4)
    g1m = _rep_blockdiag(
        jnp.concatenate([pws[0][0:16], jnp.zeros((16, 32), dtype=F32)]), r4)
    g2m = _rep_blockdiag(
        jnp.concatenate([pws[0][16:32], jnp.zeros((16, 32), dtype=F32)]), r4)
    g3m = _rep_blockdiag(pws[0][32:48], r4)
    v_b0 = _rep_row(pbs[0], r4)
    w1r = _rep_blockdiag(pws[1], r4)
    v_b1 = _rep_row(pbs[1], r4)
    wc2r = _rep_blockdiag(pws[2], r4)
    v_b2 = _rep_row(pbs[2], r4)
    m_j2 = m_j1
    v_g2 = _rep_row(pln[0], r4)
    v_be2 = _rep_row(pln[1], r4)
    pair = jnp.zeros((8, 4), dtype=F32)
    dup = jnp.zeros((4, 8), dtype=F32)
    sel = jnp.zeros((128, 4), dtype=F32)
    for q in range(4):
        pair = pair.at[2 * q, q].set(1.0).at[2 * q + 1, q].set(1.0)
        dup = dup.at[q, 2 * q].set(1.0).at[q, 2 * q + 1].set(1.0)
        sel = sel.at[32 * q + 16, q].set(1.0)
    ex1 = jnp.zeros((64, 128), dtype=F32)
    ex2 = jnp.zeros((8, 128), dtype=F32)
    for q in range(4):
        for i in range(16):
            ex1 = ex1.at[16 * q + i, 32 * q + i].set(1.0)
        for i in range(2):
            ex2 = ex2.at[2 * q + i, 32 * q + 16 + i].set(1.0)

    mats = [m_aea, m_adi, m_ags, m_agd, v_ba, m_b, v_bb, m_ce, v_bce,
            m_cw, v_bcw, m_j1, v_g1, v_be1, g1m, g2m, g3m, v_b0, w1r,
            v_b1, wc2r, v_b2, m_j2, v_g2, v_be2, pair, dup, sel, ex1, ex2]

    gs_p = gsrc.reshape(rows, 128)
    gd_p = gdst.reshape(rows, 128)
    ea4 = ea_pad.reshape(rows, 8)

    def dot(a, bm):
        return jnp.dot(a, bm, preferred_element_type=F32, precision=_PREC)

    def body(gs_ref, gd_ref, ea_ref, *rest):
        pay_ref, e2_ref = rest[-2], rest[-1]
        (mAea, mAdi, mAgs, mAgd, bA, mB, bB, mCe, bCe, mCw, bCw, j1, g1,
         be1, G1, G2, G3, b0, W1, b1, WC2, b2, j2, g2, be2, PAIR, DUP,
         SEL, EX1, EX2) = [r[...] for r in rest[:-2]]
        gs = gs_ref[...]
        gd = gd_ref[...]
        ea = ea_ref[...]
        sq = ea * ea
        d2 = dot(sq, PAIR)
        dist4 = jnp.sqrt(d2)
        inv4 = 1.0 / d2
        h = dot(ea, mAea) + dot(dist4, mAdi) + dot(gs, mAgs) + dot(gd, mAgd) + bA
        h = jnp.maximum(h, 0.0)
        h = jnp.maximum(dot(h, mB) + bB, 0.0)
        xc = dot(h, mCe) + bCe
        wp = dot(h, mCw) + bCw
        sb = dot(xc * xc, j1)
        e1 = xc / jnp.sqrt(sb + 1e-5) * g1 + be1
        dmf4 = dot(gs - gd, SEL)
        le = dot(dmf4 * inv4, DUP) * ea * wp
        hp = jnp.maximum(dot(gs, G1) + dot(gd, G2) + dot(e1, G3) + b0, 0.0)
        hp = jnp.maximum(dot(hp, W1) + b1, 0.0)
        xc2 = dot(hp, WC2) + b2
        sb2 = dot(xc2 * xc2, j2)
        eu1 = xc2 / jnp.sqrt(sb2 + 1e-5) * g2 + be2
        pay_ref[...] = dot(eu1, EX1) + dot(le, EX2)
        e2_ref[...] = e1 + eu1

    pay, e2 = pl.pallas_call(
        body,
        grid=(rows // rb,),
        in_specs=[pl.BlockSpec((rb, 128), lambda i: (i, 0)),
                  pl.BlockSpec((rb, 128), lambda i: (i, 0)),
                  pl.BlockSpec((rb, 8), lambda i: (i, 0))]
                 + [_full_spec(m) for m in mats],
        out_specs=(pl.BlockSpec((rb, 128), lambda i: (i, 0)),
                   pl.BlockSpec((rb, 64), lambda i: (i, 0))),
        out_shape=(jax.ShapeDtypeStruct((rows, 128), F32),
                   jax.ShapeDtypeStruct((rows, 64), F32)),
    )(gs_p, gd_p, ea4, *mats)
    return pay.reshape(e_pad, 32), e2.reshape(e_pad, 16)


def _blockdiag(mats):
    k_tot = sum(m.shape[0] for m in mats)
    n_tot = sum(m.shape[1] for m in mats)
    out = jnp.zeros((k_tot, n_tot), dtype=F32)
    r = c = 0
    for m in mats:
        out = out.at[r:r + m.shape[0], c:c + m.shape[1]].set(m)
        r += m.shape[0]
        c += m.shape[1]
    return out


def _tc_node_pass1(partials, t1, procn_w):
    n_pad = t1.shape[0]
    b = 1024
    ws, bs, ln = procn_w
    n_layers = len(ws)
    wb = _flatten_wb(ws, bs, ln)

    def body(p_ref, t1_ref, *rest):
        h2_ref, lc_ref = rest[-2], rest[-1]
        ws_, bs_, ln_, _ = _unflatten_wb([r[...] for r in rest[:-2]], n_layers, True)
        pblk = p_ref[...]
        agg = pblk[0] + pblk[1]
        h1 = t1_ref[...][:, 0:16]
        nu_in = jnp.concatenate([h1, agg[:, 0:16]], axis=1)
        hu = _mlp(nu_in, ws_, bs_, ln_)
        h2_ref[...] = h1 + hu
        lc_ref[...] = agg[:, 16:18]

    return pl.pallas_call(
        body,
        grid=(n_pad // b,),
        in_specs=[pl.BlockSpec((2, b, 32), lambda i: (0, i, 0)),
                  pl.BlockSpec((b, 32), lambda i: (i, 0))]
                 + [_full_spec(w) for w in wb],
        out_specs=(pl.BlockSpec((b, 16), lambda i: (i, 0)),
                   pl.BlockSpec((b, 2), lambda i: (i, 0))),
        out_shape=(jax.ShapeDtypeStruct((n_pad, 16), F32),
                   jax.ShapeDtypeStruct((n_pad, 2), F32)),
    )(partials, t1, *wb)


def _tc_edge_pass2(g2s, g2d, e2, proce_w):
    e_pad = g2s.shape[0]
    rows = e_pad // 8
    rb = 256
    ws, bs, ln = proce_w
    r8 = 8
    h1m = _rep_blockdiag(ws[0][0:16], r8)
    h2m = _rep_blockdiag(ws[0][16:32], r8)
    h3m = _rep_blockdiag(ws[0][32:48], r8)
    v_b0 = _rep_row(bs[0], r8)
    w1r = _rep_blockdiag(ws[1], r8)
    v_b1 = _rep_row(bs[1], r8)
    w2r = _rep_blockdiag(ws[2], r8)
    v_b2 = _rep_row(bs[2], r8)
    jm = _rep_blockdiag(jnp.full((16, 16), 1.0 / 16.0, dtype=F32), r8)
    v_g = _rep_row(ln[0], r8)
    v_be = _rep_row(ln[1], r8)
    mats = [h1m, h2m, h3m, v_b0, w1r, v_b1, w2r, v_b2, jm, v_g, v_be]

    def dot(a, bm):
        return jnp.dot(a, bm, preferred_element_type=F32, precision=_PREC)

    def body(gs_ref, gd_ref, e2_ref, *rest):
        o_ref = rest[-1]
        (H1, H2, H3, b0, W1, b1, W2, b2, J, g, be) = [r[...] for r in rest[:-1]]
        hp = jnp.maximum(
            dot(gs_ref[...], H1) + dot(gd_ref[...], H2)
            + dot(e2_ref[...], H3) + b0, 0.0)
        hp = jnp.maximum(dot(hp, W1) + b1, 0.0)
        xc = dot(hp, W2) + b2
        sb = dot(xc * xc, J)
        o_ref[...] = xc / jnp.sqrt(sb + 1e-5) * g + be

    out = pl.pallas_call(
        body,
        grid=(rows // rb,),
        in_specs=[pl.BlockSpec((rb, 128), lambda i: (i, 0)),
                  pl.BlockSpec((rb, 128), lambda i: (i, 0)),
                  pl.BlockSpec((rb, 128), lambda i: (i, 0))]
                 + [_full_spec(m) for m in mats],
        out_specs=pl.BlockSpec((rb, 128), lambda i: (i, 0)),
        out_shape=jax.ShapeDtypeStruct((rows, 128), F32),
    )(g2s.reshape(rows, 128), g2d.reshape(rows, 128),
      e2.reshape(rows, 128), *mats)
    return out.reshape(e_pad, 16)


def _tc_node_pass2_dec(partials2, h2, lc, procn_w, dec_w):
    n_pad = h2.shape[0]
    b = 1024
    pws, pbs, pln = procn_w
    dws, dbs, _ = dec_w
    n_p, n_d = len(pws), len(dws)
    wb = _flatten_wb(pws, pbs, pln) + _flatten_wb(dws, dbs, None)

    def body(p_ref, h2_ref, lc_ref, *rest):
        o_ref = rest[-1]
        vals = [r[...] for r in rest[:-1]]
        pws_, pbs_, pln_, vals = _unflatten_wb(vals, n_p, True)
        dws_, dbs_, _, vals = _unflatten_wb(vals, n_d, False)
        pblk = p_ref[...]
        agg = pblk[0] + pblk[1]
        h2b = h2_ref[...]
        hu = _mlp(jnp.concatenate([h2b, agg], axis=1), pws_, pbs_, pln_)
        h3 = h2b + hu
        dec_in = jnp.concatenate([lc_ref[...], h3], axis=1)
        o_ref[...] = _mlp(dec_in, dws_, dbs_, None)

    return pl.pallas_call(
        body,
        grid=(n_pad // b,),
        in_specs=[pl.BlockSpec((2, b, 16), lambda i: (0, i, 0)),
                  pl.BlockSpec((b, 16), lambda i: (i, 0)),
                  pl.BlockSpec((b, 2), lambda i: (i, 0))]
                 + [_full_spec(w) for w in wb],
        out_specs=pl.BlockSpec((b, 2), lambda i: (i, 0)),
        out_shape=jax.ShapeDtypeStruct((n_pad, 2), F32),
    )(partials2, h2, lc, *wb)


def kernel(x, edge_index, edge_attr, params):
    n = x.shape[0]
    e = edge_index.shape[1]
    n_pad = _ceil_to(n + 1, 1024)
    e_pad = _ceil_to(e, _NW * _CH)

    row = edge_index[0]
    col = edge_index[1]
    pad_idx = jnp.full((e_pad - e,), n, dtype=I32)
    row_p = jnp.concatenate([row, pad_idx]).reshape(e_pad // 128, 128)
    col_p = jnp.concatenate([col, pad_idx]).reshape(e_pad // 128, 128)
    ea_pad = jnp.concatenate(
        [edge_attr, jnp.full((e_pad - e, 2), 0.5, dtype=F32)])
    x_pad = jnp.concatenate([x, jnp.zeros((n_pad - n, x.shape[1]), dtype=F32)])

    light_w = _prep_params(params["light_edge"])
    encn_w = _prep_params(params["enc_node"])
    ence_w = _prep_params(params["enc_edge"])
    proce_w = _prep_params(params["proc_edge"])
    procn_w = _prep_params(params["proc_node"])
    dec_w = _prep_params(params["dec"])

    zeros32 = jnp.zeros((n_pad // _NS, 32), dtype=F32)
    zeros16 = jnp.zeros((n_pad // _NS, 16), dtype=F32)

    mean_x, std_x = _tc_node_stats(x)
    mean_e, std_e = _tc_edge_stats(edge_attr)

    t1 = _tc_encode_nodes(x_pad, mean_x, std_x, encn_w)

    gsrc, gdst = _sc_gather(t1, row_p, col_p)
    pay1, e2 = _tc_edge_pass1(gsrc, gdst, ea_pad, mean_e, std_e,
                              light_w, ence_w, proce_w)
    part1 = _sc_scatter_add(pay1, col_p, n_pad, zeros32, ch=256)
    h2, lc = _tc_node_pass1(part1.reshape(2, n_pad, 32), t1, procn_w)

    g2s, g2d = _sc_gather(h2, row_p, col_p)
    eu2 = _tc_edge_pass2(g2s, g2d, e2, proce_w)
    part2 = _sc_scatter_add(eu2, col_p, n_pad, zeros16, ch=1024)
    out = _tc_node_pass2_dec(part2.reshape(2, n_pad, 16), h2, lc,
                             procn_w, dec_w)
    return out[:n]
